# baseline (speedup 1.0000x reference)
import functools

import jax
import jax.numpy as jnp
from jax import lax
from jax.experimental import pallas as pl
from jax.experimental.pallas import tpu as pltpu
from jax.experimental.pallas import tpu_sc as plsc

N = 8192
M = 32768
C = 64
CS = 64
DOUT = 128
KNN = 3

BM = 128
BC = 512

_NC = 2
_NS = 16
_NW = _NC * _NS
_GCHUNK = 128
_ROWS = KNN * M
_ROWS_PER_W = _ROWS // _NW
_NCHUNK = _ROWS_PER_W // _GCHUNK


def _knn_body(q_ref, qn_ref, pt3_ref, pn3_ref, idx_ref, w_ref):
    q = q_ref[...]
    qn = qn_ref[...]
    big = jnp.float32(3.0e38)
    init = jnp.full((BM, 128), big, jnp.float32)

    def step(s, ms):
        m1, m2, m3 = ms
        qp = jnp.dot(q, pt3_ref[s], preferred_element_type=jnp.float32)
        d2 = jnp.maximum(qn + pn3_ref[s] - 2.0 * qp, 0.0)
        bs = lax.bitcast_convert_type(d2, jnp.int32)
        ks = lax.bitcast_convert_type(((bs & ~0x3F) | s) + 0x00800000,
                                      jnp.float32)
        t = jnp.maximum(m1, ks)
        m1 = jnp.minimum(m1, ks)
        u = jnp.maximum(m2, t)
        m2 = jnp.minimum(m2, t)
        m3 = jnp.minimum(m3, u)
        return (m1, m2, m3)

    m1, m2, m3 = lax.fori_loop(0, N // 128, step, (init, init, init))
    cands = jnp.concatenate([m1, m2, m3], axis=1)
    lane = lax.broadcasted_iota(jnp.int32, (BM, 3 * 128), 1)
    outs, cols = [], []
    work = cands
    for _ in range(KNN):
        mn = jnp.min(work, axis=1, keepdims=True)
        pos = jnp.where(work == mn, lane, jnp.int32(1 << 20))
        pk = jnp.min(pos, axis=1, keepdims=True)
        work = jnp.where(lane == pk, big, work)
        outs.append(mn)
        cols.append(pk & 127)
    wb = (lax.bitcast_convert_type(jnp.concatenate(outs, axis=1), jnp.int32)
          - 0x00800000)
    d2k = lax.bitcast_convert_type(wb & ~0x3F, jnp.float32)
    idx_ref[...] = (wb & 0x3F) * 128 + jnp.concatenate(cols, axis=1)
    w = 1.0 / jnp.maximum(d2k, 1e-16)
    w_ref[...] = w / jnp.sum(w, axis=1, keepdims=True)


_knn_call = pl.pallas_call(
    _knn_body,
    grid=(M // BM,),
    in_specs=[
        pl.BlockSpec((BM, 8), lambda i: (i, 0)),
        pl.BlockSpec((BM, 1), lambda i: (i, 0)),
        pl.BlockSpec((N // 128, 8, 128), lambda i: (0, 0, 0)),
        pl.BlockSpec((N // 128, 1, 128), lambda i: (0, 0, 0)),
    ],
    out_specs=[
        pl.BlockSpec((BM, KNN), lambda i: (i, 0)),
        pl.BlockSpec((BM, KNN), lambda i: (i, 0)),
    ],
    out_shape=[
        jax.ShapeDtypeStruct((M, KNN), jnp.int32),
        jax.ShapeDtypeStruct((M, KNN), jnp.float32),
    ],
)


def _sc_gather_body(idx_hbm, tab_hbm, out_hbm, idx_v, rows_v, sem):
    wid = lax.axis_index("s") * _NC + lax.axis_index("c")
    base = wid * _ROWS_PER_W

    def chunk(c, carry):
        off = base + c * _GCHUNK
        pltpu.sync_copy(idx_hbm.at[pl.ds(off, _GCHUNK)], idx_v)
        pltpu.async_copy(tab_hbm.at[idx_v], rows_v, sem).wait()
        pltpu.sync_copy(rows_v, out_hbm.at[pl.ds(off, _GCHUNK)])
        return carry

    lax.fori_loop(0, _NCHUNK, chunk, 0)


@functools.lru_cache(maxsize=None)
def _sc_gather():
    return pl.kernel(
        _sc_gather_body,
        out_type=jax.ShapeDtypeStruct((_ROWS, C), jnp.float32),
        mesh=plsc.VectorSubcoreMesh(core_axis_name="c", subcore_axis_name="s",
                                    num_cores=_NC, num_subcores=_NS),
        scratch_types=[
            pltpu.VMEM((_GCHUNK,), jnp.int32),
            pltpu.VMEM((_GCHUNK, C), jnp.float32),
            pltpu.SemaphoreType.DMA,
        ],
        compiler_params=pltpu.CompilerParams(use_tc_tiling_on_sc=False),
    )


def _mlp_body(w_ref, g0_ref, g1_ref, g2_ref, xs_ref, w1t_ref, w2t_ref, b_ref,
              y_ref):
    w = w_ref[...]
    xi = (w[:, 0:1] * g0_ref[...] + w[:, 1:2] * g1_ref[...]
          + w[:, 2:3] * g2_ref[...])
    acc = jnp.dot(xi, w1t_ref[...], preferred_element_type=jnp.float32)
    acc = acc + jnp.dot(xs_ref[...], w2t_ref[...],
                        preferred_element_type=jnp.float32)
    y_ref[...] = jnp.maximum(acc + b_ref[...], 0.0)


_mlp_call = pl.pallas_call(
    _mlp_body,
    grid=(M // BC,),
    in_specs=[
        pl.BlockSpec((BC, KNN), lambda i: (i, 0)),
        pl.BlockSpec((BC, C), lambda i: (i, 0)),
        pl.BlockSpec((BC, C), lambda i: (i + M // BC, 0)),
        pl.BlockSpec((BC, C), lambda i: (i + 2 * (M // BC), 0)),
        pl.BlockSpec((BC, CS), lambda i: (i, 0)),
        pl.BlockSpec((C, DOUT), lambda i: (0, 0)),
        pl.BlockSpec((CS, DOUT), lambda i: (0, 0)),
        pl.BlockSpec((1, DOUT), lambda i: (0, 0)),
    ],
    out_specs=pl.BlockSpec((BC, DOUT), lambda i: (i, 0)),
    out_shape=jax.ShapeDtypeStruct((M, DOUT), jnp.float32),
)


def kernel(x, pos, batch, x_skip, pos_skip, batch_skip, W, b):
    qn = jnp.sum(pos_skip * pos_skip, axis=1, keepdims=True)
    pn = jnp.sum(pos * pos, axis=1)[None, :]
    q_pad = jnp.concatenate(
        [pos_skip, jnp.zeros((M, 5), jnp.float32)], axis=1)
    pt_pad = jnp.concatenate(
        [pos.T, jnp.zeros((5, N), jnp.float32)], axis=0)
    pt3 = pt_pad.reshape(8, N // 128, 128).transpose(1, 0, 2)
    pn3 = pn.reshape(1, N // 128, 128).transpose(1, 0, 2)

    idx, w = _knn_call(q_pad, qn, pt3, pn3)

    flat_idx = idx.T.reshape(-1)
    g = _sc_gather()(flat_idx, x)

    w1t = W[:, :C].T
    w2t = W[:, C:].T
    y = _mlp_call(w, g, g, g, x_skip, w1t, w2t, b[None, :])
    return (y, pos_skip, batch_skip)


# speedup vs baseline: 3.1125x; 3.1125x over previous
import functools

import jax
import jax.numpy as jnp
from jax import lax
from jax.experimental import pallas as pl
from jax.experimental.pallas import tpu as pltpu
from jax.experimental.pallas import tpu_sc as plsc

N = 8192
M = 32768
C = 64
CS = 64
DOUT = 128
KNN = 3

BM = 128
BC = 512

_NC = 2
_NS = 16
_NW = _NC * _NS
_GCHUNK = 128
_ROWS = KNN * M
_ROWS_PER_W = _ROWS // _NW
_NCHUNK = _ROWS_PER_W // _GCHUNK


def _knn_body(q_ref, qn_ref, pt_ref, pn_ref, idx_ref, w_ref):
    qp = jnp.dot(q_ref[...], pt_ref[...], preferred_element_type=jnp.float32)
    d2 = jnp.maximum(qn_ref[...] + pn_ref[...] - 2.0 * qp, 0.0)
    bits = lax.bitcast_convert_type(d2, jnp.int32)
    big = jnp.float32(3.0e38)
    m1 = jnp.full((BM, 128), big, jnp.float32)
    m2 = m1
    m3 = m1
    for s in range(N // 128):
        ks = lax.bitcast_convert_type(
            ((bits[:, s * 128:(s + 1) * 128] & ~0x3F) | s) + 0x00800000,
            jnp.float32)
        t = jnp.maximum(m1, ks)
        m1 = jnp.minimum(m1, ks)
        u = jnp.maximum(m2, t)
        m2 = jnp.minimum(m2, t)
        m3 = jnp.minimum(m3, u)
    cands = jnp.concatenate([m1, m2, m3], axis=1)
    lane = lax.broadcasted_iota(jnp.int32, (BM, 3 * 128), 1)
    outs, cols = [], []
    work = cands
    for _ in range(KNN):
        mn = jnp.min(work, axis=1, keepdims=True)
        pos = jnp.where(work == mn, lane, jnp.int32(1 << 20))
        pk = jnp.min(pos, axis=1, keepdims=True)
        work = jnp.where(lane == pk, big, work)
        outs.append(mn)
        cols.append(pk & 127)
    wb = (lax.bitcast_convert_type(jnp.concatenate(outs, axis=1), jnp.int32)
          - 0x00800000)
    d2k = lax.bitcast_convert_type(wb & ~0x3F, jnp.float32)
    idx_ref[...] = (wb & 0x3F) * 128 + jnp.concatenate(cols, axis=1)
    w = 1.0 / jnp.maximum(d2k, 1e-16)
    w_ref[...] = w / jnp.sum(w, axis=1, keepdims=True)


_knn_call = pl.pallas_call(
    _knn_body,
    grid=(M // BM,),
    in_specs=[
        pl.BlockSpec((BM, 8), lambda i: (i, 0)),
        pl.BlockSpec((BM, 1), lambda i: (i, 0)),
        pl.BlockSpec((8, N), lambda i: (0, 0)),
        pl.BlockSpec((1, N), lambda i: (0, 0)),
    ],
    out_specs=[
        pl.BlockSpec((BM, KNN), lambda i: (i, 0)),
        pl.BlockSpec((BM, KNN), lambda i: (i, 0)),
    ],
    out_shape=[
        jax.ShapeDtypeStruct((M, KNN), jnp.int32),
        jax.ShapeDtypeStruct((M, KNN), jnp.float32),
    ],
)


def _sc_gather_body(idx_hbm, tab_hbm, out_hbm, idx_v, rows_v, sem):
    wid = lax.axis_index("s") * _NC + lax.axis_index("c")
    base = wid * _ROWS_PER_W

    def chunk(c, carry):
        off = base + c * _GCHUNK
        pltpu.sync_copy(idx_hbm.at[pl.ds(off, _GCHUNK)], idx_v)
        pltpu.async_copy(tab_hbm.at[idx_v], rows_v, sem).wait()
        pltpu.sync_copy(rows_v, out_hbm.at[pl.ds(off, _GCHUNK)])
        return carry

    lax.fori_loop(0, _NCHUNK, chunk, 0)


@functools.lru_cache(maxsize=None)
def _sc_gather():
    return pl.kernel(
        _sc_gather_body,
        out_type=jax.ShapeDtypeStruct((_ROWS, C), jnp.float32),
        mesh=plsc.VectorSubcoreMesh(core_axis_name="c", subcore_axis_name="s",
                                    num_cores=_NC, num_subcores=_NS),
        scratch_types=[
            pltpu.VMEM((_GCHUNK,), jnp.int32),
            pltpu.VMEM((_GCHUNK, C), jnp.float32),
            pltpu.SemaphoreType.DMA,
        ],
        compiler_params=pltpu.CompilerParams(use_tc_tiling_on_sc=False),
    )


def _mlp_body(w_ref, g0_ref, g1_ref, g2_ref, xs_ref, w1t_ref, w2t_ref, b_ref,
              y_ref):
    w = w_ref[...]
    xi = (w[:, 0:1] * g0_ref[...] + w[:, 1:2] * g1_ref[...]
          + w[:, 2:3] * g2_ref[...])
    acc = jnp.dot(xi, w1t_ref[...], preferred_element_type=jnp.float32)
    acc = acc + jnp.dot(xs_ref[...], w2t_ref[...],
                        preferred_element_type=jnp.float32)
    y_ref[...] = jnp.maximum(acc + b_ref[...], 0.0)


_mlp_call = pl.pallas_call(
    _mlp_body,
    grid=(M // BC,),
    in_specs=[
        pl.BlockSpec((BC, KNN), lambda i: (i, 0)),
        pl.BlockSpec((BC, C), lambda i: (i, 0)),
        pl.BlockSpec((BC, C), lambda i: (i + M // BC, 0)),
        pl.BlockSpec((BC, C), lambda i: (i + 2 * (M // BC), 0)),
        pl.BlockSpec((BC, CS), lambda i: (i, 0)),
        pl.BlockSpec((C, DOUT), lambda i: (0, 0)),
        pl.BlockSpec((CS, DOUT), lambda i: (0, 0)),
        pl.BlockSpec((1, DOUT), lambda i: (0, 0)),
    ],
    out_specs=pl.BlockSpec((BC, DOUT), lambda i: (i, 0)),
    out_shape=jax.ShapeDtypeStruct((M, DOUT), jnp.float32),
)


def kernel(x, pos, batch, x_skip, pos_skip, batch_skip, W, b):
    qn = jnp.sum(pos_skip * pos_skip, axis=1, keepdims=True)
    pn = jnp.sum(pos * pos, axis=1)[None, :]
    q_pad = jnp.concatenate(
        [pos_skip, jnp.zeros((M, 5), jnp.float32)], axis=1)
    pt_pad = jnp.concatenate(
        [pos.T, jnp.zeros((5, N), jnp.float32)], axis=0)
    idx, w = _knn_call(q_pad, qn, pt_pad, pn)

    flat_idx = idx.T.reshape(-1)
    g = _sc_gather()(flat_idx, x)

    w1t = W[:, :C].T
    w2t = W[:, C:].T
    y = _mlp_call(w, g, g, g, x_skip, w1t, w2t, b[None, :])
    return (y, pos_skip, batch_skip)


# speedup vs baseline: 3.7706x; 1.2114x over previous
import functools

import jax
import jax.numpy as jnp
from jax import lax
from jax.experimental import pallas as pl
from jax.experimental.pallas import tpu as pltpu
from jax.experimental.pallas import tpu_sc as plsc

N = 8192
M = 32768
C = 64
CS = 64
DOUT = 128
KNN = 3

BM = 256
BC = 512

_NC = 2
_NS = 16
_NW = _NC * _NS
_GCHUNK = 128
_ROWS = KNN * M
_ROWS_PER_W = _ROWS // _NW
_NCHUNK = _ROWS_PER_W // _GCHUNK


def _knn_body(q_ref, qn_ref, pt_ref, pn_ref, idx_ref, w_ref):
    qp = jnp.dot(q_ref[...], pt_ref[...], preferred_element_type=jnp.float32)
    d2 = jnp.maximum(qn_ref[...] + pn_ref[...] - 2.0 * qp, 0.0)
    bits = lax.bitcast_convert_type(d2, jnp.int32) & ~0x3F
    big = jnp.float32(3.0e38)
    m1 = jnp.full((BM, 128), big, jnp.float32)
    m2 = m1
    m3 = m1
    for s in range(N // 128):
        ks = lax.bitcast_convert_type(
            bits[:, s * 128:(s + 1) * 128] + (0x00800000 + s), jnp.float32)
        t = jnp.maximum(m1, ks)
        m1 = jnp.minimum(m1, ks)
        u = jnp.maximum(m2, t)
        m2 = jnp.minimum(m2, t)
        m3 = jnp.minimum(m3, u)
    cands = jnp.concatenate([m1, m2, m3], axis=1)
    lane = lax.broadcasted_iota(jnp.int32, (BM, 3 * 128), 1)
    outs, cols = [], []
    work = cands
    for _ in range(KNN):
        mn = jnp.min(work, axis=1, keepdims=True)
        pos = jnp.where(work == mn, lane, jnp.int32(1 << 20))
        pk = jnp.min(pos, axis=1, keepdims=True)
        work = jnp.where(lane == pk, big, work)
        outs.append(mn)
        cols.append(pk & 127)
    wb = (lax.bitcast_convert_type(jnp.concatenate(outs, axis=1), jnp.int32)
          - 0x00800000)
    d2k = lax.bitcast_convert_type(wb & ~0x3F, jnp.float32)
    idx_ref[...] = (wb & 0x3F) * 128 + jnp.concatenate(cols, axis=1)
    w = 1.0 / jnp.maximum(d2k, 1e-16)
    w_ref[...] = w / jnp.sum(w, axis=1, keepdims=True)


_knn_call = pl.pallas_call(
    _knn_body,
    grid=(M // BM,),
    in_specs=[
        pl.BlockSpec((BM, 8), lambda i: (i, 0)),
        pl.BlockSpec((BM, 1), lambda i: (i, 0)),
        pl.BlockSpec((8, N), lambda i: (0, 0)),
        pl.BlockSpec((1, N), lambda i: (0, 0)),
    ],
    out_specs=[
        pl.BlockSpec((BM, KNN), lambda i: (i, 0)),
        pl.BlockSpec((BM, KNN), lambda i: (i, 0)),
    ],
    out_shape=[
        jax.ShapeDtypeStruct((M, KNN), jnp.int32),
        jax.ShapeDtypeStruct((M, KNN), jnp.float32),
    ],
)


def _sc_gather_body(idx_hbm, tab_hbm, out_hbm, idx_v, rows_v, sem):
    wid = lax.axis_index("s") * _NC + lax.axis_index("c")
    base = wid * _ROWS_PER_W

    def chunk(c, carry):
        off = base + c * _GCHUNK
        pltpu.sync_copy(idx_hbm.at[pl.ds(off, _GCHUNK)], idx_v)
        pltpu.async_copy(tab_hbm.at[idx_v], rows_v, sem).wait()
        pltpu.sync_copy(rows_v, out_hbm.at[pl.ds(off, _GCHUNK)])
        return carry

    lax.fori_loop(0, _NCHUNK, chunk, 0)


@functools.lru_cache(maxsize=None)
def _sc_gather():
    return pl.kernel(
        _sc_gather_body,
        out_type=jax.ShapeDtypeStruct((_ROWS, C), jnp.float32),
        mesh=plsc.VectorSubcoreMesh(core_axis_name="c", subcore_axis_name="s",
                                    num_cores=_NC, num_subcores=_NS),
        scratch_types=[
            pltpu.VMEM((_GCHUNK,), jnp.int32),
            pltpu.VMEM((_GCHUNK, C), jnp.float32),
            pltpu.SemaphoreType.DMA,
        ],
        compiler_params=pltpu.CompilerParams(use_tc_tiling_on_sc=False),
    )


def _mlp_body(w_ref, g0_ref, g1_ref, g2_ref, xs_ref, w1t_ref, w2t_ref, b_ref,
              y_ref):
    w = w_ref[...]
    xi = (w[:, 0:1] * g0_ref[...] + w[:, 1:2] * g1_ref[...]
          + w[:, 2:3] * g2_ref[...])
    acc = jnp.dot(xi, w1t_ref[...], preferred_element_type=jnp.float32)
    acc = acc + jnp.dot(xs_ref[...], w2t_ref[...],
                        preferred_element_type=jnp.float32)
    y_ref[...] = jnp.maximum(acc + b_ref[...], 0.0)


_mlp_call = pl.pallas_call(
    _mlp_body,
    grid=(M // BC,),
    in_specs=[
        pl.BlockSpec((BC, KNN), lambda i: (i, 0)),
        pl.BlockSpec((BC, C), lambda i: (i, 0)),
        pl.BlockSpec((BC, C), lambda i: (i + M // BC, 0)),
        pl.BlockSpec((BC, C), lambda i: (i + 2 * (M // BC), 0)),
        pl.BlockSpec((BC, CS), lambda i: (i, 0)),
        pl.BlockSpec((C, DOUT), lambda i: (0, 0)),
        pl.BlockSpec((CS, DOUT), lambda i: (0, 0)),
        pl.BlockSpec((1, DOUT), lambda i: (0, 0)),
    ],
    out_specs=pl.BlockSpec((BC, DOUT), lambda i: (i, 0)),
    out_shape=jax.ShapeDtypeStruct((M, DOUT), jnp.float32),
)


def kernel(x, pos, batch, x_skip, pos_skip, batch_skip, W, b):
    qn = jnp.sum(pos_skip * pos_skip, axis=1, keepdims=True)
    pn = jnp.sum(pos * pos, axis=1)[None, :]
    q_pad = jnp.concatenate(
        [pos_skip, jnp.zeros((M, 5), jnp.float32)], axis=1)
    pt_pad = jnp.concatenate(
        [pos.T, jnp.zeros((5, N), jnp.float32)], axis=0)
    idx, w = _knn_call(q_pad, qn, pt_pad, pn)

    flat_idx = idx.T.reshape(-1)
    g = _sc_gather()(flat_idx, x)

    w1t = W[:, :C].T
    w2t = W[:, C:].T
    y = _mlp_call(w, g, g, g, x_skip, w1t, w2t, b[None, :])
    return (y, pos_skip, batch_skip)


# speedup vs baseline: 4.1023x; 1.0880x over previous
import functools

import jax
import jax.numpy as jnp
from jax import lax
from jax.experimental import pallas as pl
from jax.experimental.pallas import tpu as pltpu
from jax.experimental.pallas import tpu_sc as plsc

N = 8192
M = 32768
C = 64
CS = 64
DOUT = 128
KNN = 3

BM = 512
BC = 512

_NC = 2
_NS = 16
_NW = _NC * _NS
_GCHUNK = 128
_ROWS = KNN * M
_ROWS_PER_W = _ROWS // _NW
_NCHUNK = _ROWS_PER_W // _GCHUNK


def _knn_body(q_ref, qn_ref, pt_ref, pn_ref, idx_ref, w_ref):
    qp = jnp.dot(q_ref[...], pt_ref[...], preferred_element_type=jnp.float32)
    d2 = jnp.maximum(qn_ref[...] + pn_ref[...] - 2.0 * qp, 0.0)
    bits = lax.bitcast_convert_type(d2, jnp.int32) & ~0x3F
    big = jnp.float32(3.0e38)
    m1 = jnp.full((BM, 128), big, jnp.float32)
    m2 = m1
    m3 = m1
    for s in range(N // 128):
        ks = lax.bitcast_convert_type(
            bits[:, s * 128:(s + 1) * 128] + (0x00800000 + s), jnp.float32)
        t = jnp.maximum(m1, ks)
        m1 = jnp.minimum(m1, ks)
        u = jnp.maximum(m2, t)
        m2 = jnp.minimum(m2, t)
        m3 = jnp.minimum(m3, u)
    cands = jnp.concatenate([m1, m2, m3], axis=1)
    lane = lax.broadcasted_iota(jnp.int32, (BM, 3 * 128), 1)
    outs, cols = [], []
    work = cands
    for _ in range(KNN):
        mn = jnp.min(work, axis=1, keepdims=True)
        pos = jnp.where(work == mn, lane, jnp.int32(1 << 20))
        pk = jnp.min(pos, axis=1, keepdims=True)
        work = jnp.where(lane == pk, big, work)
        outs.append(mn)
        cols.append(pk & 127)
    wb = (lax.bitcast_convert_type(jnp.concatenate(outs, axis=1), jnp.int32)
          - 0x00800000)
    d2k = lax.bitcast_convert_type(wb & ~0x3F, jnp.float32)
    idx_ref[...] = (wb & 0x3F) * 128 + jnp.concatenate(cols, axis=1)
    w = 1.0 / jnp.maximum(d2k, 1e-16)
    w_ref[...] = w / jnp.sum(w, axis=1, keepdims=True)


_knn_call = pl.pallas_call(
    _knn_body,
    grid=(M // BM,),
    in_specs=[
        pl.BlockSpec((BM, 8), lambda i: (i, 0)),
        pl.BlockSpec((BM, 1), lambda i: (i, 0)),
        pl.BlockSpec((8, N), lambda i: (0, 0)),
        pl.BlockSpec((1, N), lambda i: (0, 0)),
    ],
    out_specs=[
        pl.BlockSpec((BM, KNN), lambda i: (i, 0)),
        pl.BlockSpec((BM, KNN), lambda i: (i, 0)),
    ],
    out_shape=[
        jax.ShapeDtypeStruct((M, KNN), jnp.int32),
        jax.ShapeDtypeStruct((M, KNN), jnp.float32),
    ],
)


def _sc_gather_body(idx_hbm, tab_hbm, out_hbm, idx_v, rows_v, sem):
    wid = lax.axis_index("s") * _NC + lax.axis_index("c")
    base = wid * _ROWS_PER_W

    def chunk(c, carry):
        off = base + c * _GCHUNK
        pltpu.sync_copy(idx_hbm.at[pl.ds(off, _GCHUNK)], idx_v)
        pltpu.async_copy(tab_hbm.at[idx_v], rows_v, sem).wait()
        pltpu.sync_copy(rows_v, out_hbm.at[pl.ds(off, _GCHUNK)])
        return carry

    lax.fori_loop(0, _NCHUNK, chunk, 0)


@functools.lru_cache(maxsize=None)
def _sc_gather():
    return pl.kernel(
        _sc_gather_body,
        out_type=jax.ShapeDtypeStruct((_ROWS, C), jnp.float32),
        mesh=plsc.VectorSubcoreMesh(core_axis_name="c", subcore_axis_name="s",
                                    num_cores=_NC, num_subcores=_NS),
        scratch_types=[
            pltpu.VMEM((_GCHUNK,), jnp.int32),
            pltpu.VMEM((_GCHUNK, C), jnp.float32),
            pltpu.SemaphoreType.DMA,
        ],
        compiler_params=pltpu.CompilerParams(use_tc_tiling_on_sc=False),
    )


def _mlp_body(w_ref, g0_ref, g1_ref, g2_ref, xs_ref, w1t_ref, w2t_ref, b_ref,
              y_ref):
    w = w_ref[...]
    xi = (w[:, 0:1] * g0_ref[...] + w[:, 1:2] * g1_ref[...]
          + w[:, 2:3] * g2_ref[...])
    acc = jnp.dot(xi, w1t_ref[...], preferred_element_type=jnp.float32)
    acc = acc + jnp.dot(xs_ref[...], w2t_ref[...],
                        preferred_element_type=jnp.float32)
    y_ref[...] = jnp.maximum(acc + b_ref[...], 0.0)


_mlp_call = pl.pallas_call(
    _mlp_body,
    grid=(M // BC,),
    in_specs=[
        pl.BlockSpec((BC, KNN), lambda i: (i, 0)),
        pl.BlockSpec((BC, C), lambda i: (i, 0)),
        pl.BlockSpec((BC, C), lambda i: (i + M // BC, 0)),
        pl.BlockSpec((BC, C), lambda i: (i + 2 * (M // BC), 0)),
        pl.BlockSpec((BC, CS), lambda i: (i, 0)),
        pl.BlockSpec((C, DOUT), lambda i: (0, 0)),
        pl.BlockSpec((CS, DOUT), lambda i: (0, 0)),
        pl.BlockSpec((1, DOUT), lambda i: (0, 0)),
    ],
    out_specs=pl.BlockSpec((BC, DOUT), lambda i: (i, 0)),
    out_shape=jax.ShapeDtypeStruct((M, DOUT), jnp.float32),
)


def kernel(x, pos, batch, x_skip, pos_skip, batch_skip, W, b):
    qn = jnp.sum(pos_skip * pos_skip, axis=1, keepdims=True)
    pn = jnp.sum(pos * pos, axis=1)[None, :]
    q_pad = jnp.concatenate(
        [pos_skip, jnp.zeros((M, 5), jnp.float32)], axis=1)
    pt_pad = jnp.concatenate(
        [pos.T, jnp.zeros((5, N), jnp.float32)], axis=0)
    idx, w = _knn_call(q_pad, qn, pt_pad, pn)

    flat_idx = idx.T.reshape(-1)
    g = _sc_gather()(flat_idx, x)

    w1t = W[:, :C].T
    w2t = W[:, C:].T
    y = _mlp_call(w, g, g, g, x_skip, w1t, w2t, b[None, :])
    return (y, pos_skip, batch_skip)


# speedup vs baseline: 4.3070x; 1.0499x over previous
import functools

import jax
import jax.numpy as jnp
from jax import lax
from jax.experimental import pallas as pl
from jax.experimental.pallas import tpu as pltpu
from jax.experimental.pallas import tpu_sc as plsc

N = 8192
M = 32768
C = 64
CS = 64
DOUT = 128
KNN = 3

BM = 1024
BC = 512

_NC = 2
_NS = 16
_NW = _NC * _NS
_GCHUNK = 128
_ROWS = KNN * M
_ROWS_PER_W = _ROWS // _NW
_NCHUNK = _ROWS_PER_W // _GCHUNK


def _knn_body(q_ref, qn_ref, pt_ref, pn_ref, idx_ref, w_ref):
    qp = jnp.dot(q_ref[...], pt_ref[...], preferred_element_type=jnp.float32)
    d2 = jnp.maximum(qn_ref[...] + pn_ref[...] - 2.0 * qp, 0.0)
    bits = lax.bitcast_convert_type(d2, jnp.int32) & ~0x3F
    big = jnp.float32(3.0e38)
    m1 = jnp.full((BM, 128), big, jnp.float32)
    m2 = m1
    m3 = m1
    for s in range(N // 128):
        ks = lax.bitcast_convert_type(
            bits[:, s * 128:(s + 1) * 128] + (0x00800000 + s), jnp.float32)
        t = jnp.maximum(m1, ks)
        m1 = jnp.minimum(m1, ks)
        u = jnp.maximum(m2, t)
        m2 = jnp.minimum(m2, t)
        m3 = jnp.minimum(m3, u)
    cands = jnp.concatenate([m1, m2, m3], axis=1)
    lane = lax.broadcasted_iota(jnp.int32, (BM, 3 * 128), 1)
    outs, cols = [], []
    work = cands
    for _ in range(KNN):
        mn = jnp.min(work, axis=1, keepdims=True)
        pos = jnp.where(work == mn, lane, jnp.int32(1 << 20))
        pk = jnp.min(pos, axis=1, keepdims=True)
        work = jnp.where(lane == pk, big, work)
        outs.append(mn)
        cols.append(pk & 127)
    wb = (lax.bitcast_convert_type(jnp.concatenate(outs, axis=1), jnp.int32)
          - 0x00800000)
    d2k = lax.bitcast_convert_type(wb & ~0x3F, jnp.float32)
    idx_ref[...] = (wb & 0x3F) * 128 + jnp.concatenate(cols, axis=1)
    w = 1.0 / jnp.maximum(d2k, 1e-16)
    w_ref[...] = w / jnp.sum(w, axis=1, keepdims=True)


_knn_call = pl.pallas_call(
    _knn_body,
    grid=(M // BM,),
    in_specs=[
        pl.BlockSpec((BM, 8), lambda i: (i, 0)),
        pl.BlockSpec((BM, 1), lambda i: (i, 0)),
        pl.BlockSpec((8, N), lambda i: (0, 0)),
        pl.BlockSpec((1, N), lambda i: (0, 0)),
    ],
    out_specs=[
        pl.BlockSpec((BM, KNN), lambda i: (i, 0)),
        pl.BlockSpec((BM, KNN), lambda i: (i, 0)),
    ],
    out_shape=[
        jax.ShapeDtypeStruct((M, KNN), jnp.int32),
        jax.ShapeDtypeStruct((M, KNN), jnp.float32),
    ],
)


def _sc_gather_body(idx_hbm, tab_hbm, out_hbm, idx_v, rows_v, sem):
    wid = lax.axis_index("s") * _NC + lax.axis_index("c")
    base = wid * _ROWS_PER_W

    def chunk(c, carry):
        off = base + c * _GCHUNK
        pltpu.sync_copy(idx_hbm.at[pl.ds(off, _GCHUNK)], idx_v)
        pltpu.async_copy(tab_hbm.at[idx_v], rows_v, sem).wait()
        pltpu.sync_copy(rows_v, out_hbm.at[pl.ds(off, _GCHUNK)])
        return carry

    lax.fori_loop(0, _NCHUNK, chunk, 0)


@functools.lru_cache(maxsize=None)
def _sc_gather():
    return pl.kernel(
        _sc_gather_body,
        out_type=jax.ShapeDtypeStruct((_ROWS, C), jnp.float32),
        mesh=plsc.VectorSubcoreMesh(core_axis_name="c", subcore_axis_name="s",
                                    num_cores=_NC, num_subcores=_NS),
        scratch_types=[
            pltpu.VMEM((_GCHUNK,), jnp.int32),
            pltpu.VMEM((_GCHUNK, C), jnp.float32),
            pltpu.SemaphoreType.DMA,
        ],
        compiler_params=pltpu.CompilerParams(use_tc_tiling_on_sc=False),
    )


def _mlp_body(w_ref, g0_ref, g1_ref, g2_ref, xs_ref, w1t_ref, w2t_ref, b_ref,
              y_ref):
    w = w_ref[...]
    xi = (w[:, 0:1] * g0_ref[...] + w[:, 1:2] * g1_ref[...]
          + w[:, 2:3] * g2_ref[...])
    acc = jnp.dot(xi, w1t_ref[...], preferred_element_type=jnp.float32)
    acc = acc + jnp.dot(xs_ref[...], w2t_ref[...],
                        preferred_element_type=jnp.float32)
    y_ref[...] = jnp.maximum(acc + b_ref[...], 0.0)


_mlp_call = pl.pallas_call(
    _mlp_body,
    grid=(M // BC,),
    in_specs=[
        pl.BlockSpec((BC, KNN), lambda i: (i, 0)),
        pl.BlockSpec((BC, C), lambda i: (i, 0)),
        pl.BlockSpec((BC, C), lambda i: (i + M // BC, 0)),
        pl.BlockSpec((BC, C), lambda i: (i + 2 * (M // BC), 0)),
        pl.BlockSpec((BC, CS), lambda i: (i, 0)),
        pl.BlockSpec((C, DOUT), lambda i: (0, 0)),
        pl.BlockSpec((CS, DOUT), lambda i: (0, 0)),
        pl.BlockSpec((1, DOUT), lambda i: (0, 0)),
    ],
    out_specs=pl.BlockSpec((BC, DOUT), lambda i: (i, 0)),
    out_shape=jax.ShapeDtypeStruct((M, DOUT), jnp.float32),
)


def kernel(x, pos, batch, x_skip, pos_skip, batch_skip, W, b):
    qn = jnp.sum(pos_skip * pos_skip, axis=1, keepdims=True)
    pn = jnp.sum(pos * pos, axis=1)[None, :]
    q_pad = jnp.concatenate(
        [pos_skip, jnp.zeros((M, 5), jnp.float32)], axis=1)
    pt_pad = jnp.concatenate(
        [pos.T, jnp.zeros((5, N), jnp.float32)], axis=0)
    idx, w = _knn_call(q_pad, qn, pt_pad, pn)

    flat_idx = idx.T.reshape(-1)
    g = _sc_gather()(flat_idx, x)

    w1t = W[:, :C].T
    w2t = W[:, C:].T
    y = _mlp_call(w, g, g, g, x_skip, w1t, w2t, b[None, :])
    return (y, pos_skip, batch_skip)


# speedup vs baseline: 4.4405x; 1.0310x over previous
import functools

import jax
import jax.numpy as jnp
from jax import lax
from jax.experimental import pallas as pl
from jax.experimental.pallas import tpu as pltpu
from jax.experimental.pallas import tpu_sc as plsc

N = 8192
M = 32768
C = 64
CS = 64
DOUT = 128
KNN = 3

BM = 1024
BC = 512

_NC = 2
_NS = 16
_NW = _NC * _NS
_GCHUNK = 128
_ROWS = KNN * M
_ROWS_PER_W = _ROWS // _NW
_NCHUNK = _ROWS_PER_W // _GCHUNK


def _knn_body(q_ref, qn_ref, pt_ref, pn_ref, idx_ref, w_ref):
    qp = jnp.dot(q_ref[...], pt_ref[...], preferred_element_type=jnp.float32)
    d2 = jnp.maximum(qn_ref[...] + pn_ref[...] - 2.0 * qp, 0.0)
    bits = lax.bitcast_convert_type(d2, jnp.int32) & ~0x3F
    big = jnp.float32(3.0e38)
    m1 = jnp.full((BM, 128), big, jnp.float32)
    m2 = m1
    m3 = m1
    for s in range(N // 128):
        ks = lax.bitcast_convert_type(
            bits[:, s * 128:(s + 1) * 128] + (0x00800000 + s), jnp.float32)
        t = jnp.maximum(m1, ks)
        m1 = jnp.minimum(m1, ks)
        u = jnp.maximum(m2, t)
        m2 = jnp.minimum(m2, t)
        m3 = jnp.minimum(m3, u)
    cands = jnp.concatenate([m1, m2, m3], axis=1)
    lane = lax.broadcasted_iota(jnp.int32, (BM, 3 * 128), 1)
    outs, cols = [], []
    work = cands
    for _ in range(KNN):
        mn = jnp.min(work, axis=1, keepdims=True)
        pos = jnp.where(work == mn, lane, jnp.int32(1 << 20))
        pk = jnp.min(pos, axis=1, keepdims=True)
        work = jnp.where(lane == pk, big, work)
        outs.append(mn)
        cols.append(pk & 127)
    wb = (lax.bitcast_convert_type(jnp.concatenate(outs, axis=1), jnp.int32)
          - 0x00800000)
    d2k = lax.bitcast_convert_type(wb & ~0x3F, jnp.float32)
    idx_ref[...] = (wb & 0x3F) * 128 + jnp.concatenate(cols, axis=1)
    w = 1.0 / jnp.maximum(d2k, 1e-16)
    w_ref[...] = w / jnp.sum(w, axis=1, keepdims=True)


_knn_call = pl.pallas_call(
    _knn_body,
    grid=(M // BM,),
    in_specs=[
        pl.BlockSpec((BM, 8), lambda i: (i, 0)),
        pl.BlockSpec((BM, 1), lambda i: (i, 0)),
        pl.BlockSpec((8, N), lambda i: (0, 0)),
        pl.BlockSpec((1, N), lambda i: (0, 0)),
    ],
    out_specs=[
        pl.BlockSpec((BM, KNN), lambda i: (i, 0)),
        pl.BlockSpec((BM, KNN), lambda i: (i, 0)),
    ],
    out_shape=[
        jax.ShapeDtypeStruct((M, KNN), jnp.int32),
        jax.ShapeDtypeStruct((M, KNN), jnp.float32),
    ],
)


_NB = 3


def _sc_gather_body(idx_hbm, tab_hbm, out_hbm, idx_v, rows_v,
                    si0, si1, si2, sg0, sg1, sg2, ss0, ss1, ss2):
    wid = lax.axis_index("s") * _NC + lax.axis_index("c")
    base = wid * _ROWS_PER_W
    sem_i, sem_g, sem_s = (si0, si1, si2), (sg0, sg1, sg2), (ss0, ss1, ss2)
    ci, cg, cs = {}, {}, {}

    def start_idx(c):
        b = c % _NB
        ci[c] = pltpu.async_copy(
            idx_hbm.at[pl.ds(base + c * _GCHUNK, _GCHUNK)],
            idx_v.at[b], sem_i[b])

    for c in range(min(_NB, _NCHUNK)):
        start_idx(c)
    for c in range(_NCHUNK):
        b = c % _NB
        ci[c].wait()
        if c >= _NB:
            cs[c - _NB].wait()
        cg[c] = pltpu.async_copy(tab_hbm.at[idx_v.at[b]], rows_v.at[b],
                                 sem_g[b])
        if c >= 1:
            bp = (c - 1) % _NB
            cg[c - 1].wait()
            cs[c - 1] = pltpu.async_copy(
                rows_v.at[bp],
                out_hbm.at[pl.ds(base + (c - 1) * _GCHUNK, _GCHUNK)],
                sem_s[bp])
            if c + 2 < _NCHUNK:
                start_idx(c + 2)
    c = _NCHUNK - 1
    cg[c].wait()
    cs[c] = pltpu.async_copy(
        rows_v.at[c % _NB],
        out_hbm.at[pl.ds(base + c * _GCHUNK, _GCHUNK)], sem_s[c % _NB])
    for c in range(max(0, _NCHUNK - _NB), _NCHUNK):
        cs[c].wait()


@functools.lru_cache(maxsize=None)
def _sc_gather():
    return pl.kernel(
        _sc_gather_body,
        out_type=jax.ShapeDtypeStruct((_ROWS, C), jnp.float32),
        mesh=plsc.VectorSubcoreMesh(core_axis_name="c", subcore_axis_name="s",
                                    num_cores=_NC, num_subcores=_NS),
        scratch_types=[
            pltpu.VMEM((_NB, _GCHUNK), jnp.int32),
            pltpu.VMEM((_NB, _GCHUNK, C), jnp.float32),
        ] + [pltpu.SemaphoreType.DMA] * (3 * _NB),
        compiler_params=pltpu.CompilerParams(use_tc_tiling_on_sc=False),
    )


def _mlp_body(w_ref, g0_ref, g1_ref, g2_ref, xs_ref, w1t_ref, w2t_ref, b_ref,
              y_ref):
    w = w_ref[...]
    xi = (w[:, 0:1] * g0_ref[...] + w[:, 1:2] * g1_ref[...]
          + w[:, 2:3] * g2_ref[...])
    acc = jnp.dot(xi, w1t_ref[...], preferred_element_type=jnp.float32)
    acc = acc + jnp.dot(xs_ref[...], w2t_ref[...],
                        preferred_element_type=jnp.float32)
    y_ref[...] = jnp.maximum(acc + b_ref[...], 0.0)


_mlp_call = pl.pallas_call(
    _mlp_body,
    grid=(M // BC,),
    in_specs=[
        pl.BlockSpec((BC, KNN), lambda i: (i, 0)),
        pl.BlockSpec((BC, C), lambda i: (i, 0)),
        pl.BlockSpec((BC, C), lambda i: (i + M // BC, 0)),
        pl.BlockSpec((BC, C), lambda i: (i + 2 * (M // BC), 0)),
        pl.BlockSpec((BC, CS), lambda i: (i, 0)),
        pl.BlockSpec((C, DOUT), lambda i: (0, 0)),
        pl.BlockSpec((CS, DOUT), lambda i: (0, 0)),
        pl.BlockSpec((1, DOUT), lambda i: (0, 0)),
    ],
    out_specs=pl.BlockSpec((BC, DOUT), lambda i: (i, 0)),
    out_shape=jax.ShapeDtypeStruct((M, DOUT), jnp.float32),
)


def kernel(x, pos, batch, x_skip, pos_skip, batch_skip, W, b):
    qn = jnp.sum(pos_skip * pos_skip, axis=1, keepdims=True)
    pn = jnp.sum(pos * pos, axis=1)[None, :]
    q_pad = jnp.concatenate(
        [pos_skip, jnp.zeros((M, 5), jnp.float32)], axis=1)
    pt_pad = jnp.concatenate(
        [pos.T, jnp.zeros((5, N), jnp.float32)], axis=0)
    idx, w = _knn_call(q_pad, qn, pt_pad, pn)

    flat_idx = idx.T.reshape(-1)
    g = _sc_gather()(flat_idx, x)

    w1t = W[:, :C].T
    w2t = W[:, C:].T
    y = _mlp_call(w, g, g, g, x_skip, w1t, w2t, b[None, :])
    return (y, pos_skip, batch_skip)


# speedup vs baseline: 4.6624x; 1.0500x over previous
import functools

import jax
import jax.numpy as jnp
from jax import lax
from jax.experimental import pallas as pl
from jax.experimental.pallas import tpu as pltpu
from jax.experimental.pallas import tpu_sc as plsc

N = 8192
M = 32768
C = 64
CS = 64
DOUT = 128
KNN = 3

BM = 1024
BC = 512

_NC = 2
_NS = 16
_NW = _NC * _NS
_GCHUNK = 128
_ROWS = KNN * M
_ROWS_PER_W = _ROWS // _NW
_NCHUNK = _ROWS_PER_W // _GCHUNK


def _knn_body(q_ref, qn_ref, pt_ref, pn_ref, idx_ref, w_ref):
    qp = jnp.dot(q_ref[...], pt_ref[...], preferred_element_type=jnp.float32)
    qn = qn_ref[...]
    big = jnp.float32(3.0e38)
    m1 = jnp.full((BM, 128), big, jnp.float32)
    m2 = m1
    m3 = m1
    for s in range(N // 128):
        sl = slice(s * 128, (s + 1) * 128)
        d2 = jnp.maximum((qn + pn_ref[:, sl]) + qp[:, sl], 0.0)
        ks = lax.bitcast_convert_type(
            (lax.bitcast_convert_type(d2, jnp.int32) & ~0x3F)
            + (0x00800000 + s), jnp.float32)
        t = jnp.maximum(m1, ks)
        m1 = jnp.minimum(m1, ks)
        u = jnp.maximum(m2, t)
        m2 = jnp.minimum(m2, t)
        m3 = jnp.minimum(m3, u)
    cands = jnp.concatenate([m1, m2, m3], axis=1)
    lane = lax.broadcasted_iota(jnp.int32, (BM, 3 * 128), 1)
    outs, cols = [], []
    work = cands
    for _ in range(KNN):
        mn = jnp.min(work, axis=1, keepdims=True)
        pos = jnp.where(work == mn, lane, jnp.int32(1 << 20))
        pk = jnp.min(pos, axis=1, keepdims=True)
        work = jnp.where(lane == pk, big, work)
        outs.append(mn)
        cols.append(pk & 127)
    wb = (lax.bitcast_convert_type(jnp.concatenate(outs, axis=1), jnp.int32)
          - 0x00800000)
    d2k = lax.bitcast_convert_type(wb & ~0x3F, jnp.float32)
    idx_ref[...] = (wb & 0x3F) * 128 + jnp.concatenate(cols, axis=1)
    w = 1.0 / jnp.maximum(d2k, 1e-16)
    w_ref[...] = w / jnp.sum(w, axis=1, keepdims=True)


_knn_call = pl.pallas_call(
    _knn_body,
    grid=(M // BM,),
    in_specs=[
        pl.BlockSpec((BM, 8), lambda i: (i, 0)),
        pl.BlockSpec((BM, 1), lambda i: (i, 0)),
        pl.BlockSpec((8, N), lambda i: (0, 0)),
        pl.BlockSpec((1, N), lambda i: (0, 0)),
    ],
    out_specs=[
        pl.BlockSpec((BM, KNN), lambda i: (i, 0)),
        pl.BlockSpec((BM, KNN), lambda i: (i, 0)),
    ],
    out_shape=[
        jax.ShapeDtypeStruct((M, KNN), jnp.int32),
        jax.ShapeDtypeStruct((M, KNN), jnp.float32),
    ],
)


_NB = 3


def _sc_gather_body(idx_hbm, tab_hbm, out_hbm, idx_v, rows_v,
                    si0, si1, si2, sg0, sg1, sg2, ss0, ss1, ss2):
    wid = lax.axis_index("s") * _NC + lax.axis_index("c")
    base = wid * _ROWS_PER_W
    sem_i, sem_g, sem_s = (si0, si1, si2), (sg0, sg1, sg2), (ss0, ss1, ss2)
    ci, cg, cs = {}, {}, {}

    def start_idx(c):
        b = c % _NB
        ci[c] = pltpu.async_copy(
            idx_hbm.at[pl.ds(base + c * _GCHUNK, _GCHUNK)],
            idx_v.at[b], sem_i[b])

    for c in range(min(_NB, _NCHUNK)):
        start_idx(c)
    for c in range(_NCHUNK):
        b = c % _NB
        ci[c].wait()
        if c >= _NB:
            cs[c - _NB].wait()
        cg[c] = pltpu.async_copy(tab_hbm.at[idx_v.at[b]], rows_v.at[b],
                                 sem_g[b])
        if c >= 1:
            bp = (c - 1) % _NB
            cg[c - 1].wait()
            cs[c - 1] = pltpu.async_copy(
                rows_v.at[bp],
                out_hbm.at[pl.ds(base + (c - 1) * _GCHUNK, _GCHUNK)],
                sem_s[bp])
            if c + 2 < _NCHUNK:
                start_idx(c + 2)
    c = _NCHUNK - 1
    cg[c].wait()
    cs[c] = pltpu.async_copy(
        rows_v.at[c % _NB],
        out_hbm.at[pl.ds(base + c * _GCHUNK, _GCHUNK)], sem_s[c % _NB])
    for c in range(max(0, _NCHUNK - _NB), _NCHUNK):
        cs[c].wait()


@functools.lru_cache(maxsize=None)
def _sc_gather():
    return pl.kernel(
        _sc_gather_body,
        out_type=jax.ShapeDtypeStruct((_ROWS, C), jnp.float32),
        mesh=plsc.VectorSubcoreMesh(core_axis_name="c", subcore_axis_name="s",
                                    num_cores=_NC, num_subcores=_NS),
        scratch_types=[
            pltpu.VMEM((_NB, _GCHUNK), jnp.int32),
            pltpu.VMEM((_NB, _GCHUNK, C), jnp.float32),
        ] + [pltpu.SemaphoreType.DMA] * (3 * _NB),
        compiler_params=pltpu.CompilerParams(use_tc_tiling_on_sc=False),
    )


def _mlp_body(w_ref, g0_ref, g1_ref, g2_ref, xs_ref, w1t_ref, w2t_ref, b_ref,
              y_ref):
    w = w_ref[...]
    xi = (w[:, 0:1] * g0_ref[...] + w[:, 1:2] * g1_ref[...]
          + w[:, 2:3] * g2_ref[...])
    acc = jnp.dot(xi, w1t_ref[...], preferred_element_type=jnp.float32)
    acc = acc + jnp.dot(xs_ref[...], w2t_ref[...],
                        preferred_element_type=jnp.float32)
    y_ref[...] = jnp.maximum(acc + b_ref[...], 0.0)


_mlp_call = pl.pallas_call(
    _mlp_body,
    grid=(M // BC,),
    in_specs=[
        pl.BlockSpec((BC, KNN), lambda i: (i, 0)),
        pl.BlockSpec((BC, C), lambda i: (i, 0)),
        pl.BlockSpec((BC, C), lambda i: (i + M // BC, 0)),
        pl.BlockSpec((BC, C), lambda i: (i + 2 * (M // BC), 0)),
        pl.BlockSpec((BC, CS), lambda i: (i, 0)),
        pl.BlockSpec((C, DOUT), lambda i: (0, 0)),
        pl.BlockSpec((CS, DOUT), lambda i: (0, 0)),
        pl.BlockSpec((1, DOUT), lambda i: (0, 0)),
    ],
    out_specs=pl.BlockSpec((BC, DOUT), lambda i: (i, 0)),
    out_shape=jax.ShapeDtypeStruct((M, DOUT), jnp.float32),
)


def kernel(x, pos, batch, x_skip, pos_skip, batch_skip, W, b):
    qn = jnp.sum(pos_skip * pos_skip, axis=1, keepdims=True)
    pn = jnp.sum(pos * pos, axis=1)[None, :]
    q_pad = jnp.concatenate(
        [pos_skip, jnp.zeros((M, 5), jnp.float32)], axis=1)
    pt_pad = jnp.concatenate(
        [(-2.0 * pos).T, jnp.zeros((5, N), jnp.float32)], axis=0)
    idx, w = _knn_call(q_pad, qn, pt_pad, pn)

    flat_idx = idx.T.reshape(-1)
    g = _sc_gather()(flat_idx, x)

    w1t = W[:, :C].T
    w2t = W[:, C:].T
    y = _mlp_call(w, g, g, g, x_skip, w1t, w2t, b[None, :])
    return (y, pos_skip, batch_skip)


# speedup vs baseline: 4.9061x; 1.0523x over previous
import functools

import jax
import jax.numpy as jnp
from jax import lax
from jax.experimental import pallas as pl
from jax.experimental.pallas import tpu as pltpu
from jax.experimental.pallas import tpu_sc as plsc

N = 8192
M = 32768
C = 64
CS = 64
DOUT = 128
KNN = 3

BM = 1024
BC = 2048

_NC = 2
_NS = 16
_NW = _NC * _NS
_GCHUNK = 128
_ROWS = KNN * M
_ROWS_PER_W = _ROWS // _NW
_NCHUNK = _ROWS_PER_W // _GCHUNK


def _knn_body(q_ref, qn_ref, pt_ref, pn_ref, idx_ref, w_ref):
    qp = jnp.dot(q_ref[...], pt_ref[...], preferred_element_type=jnp.float32)
    qn = qn_ref[...]
    big = jnp.float32(3.0e38)
    m1 = jnp.full((BM, 128), big, jnp.float32)
    m2 = m1
    m3 = m1
    for s in range(N // 128):
        sl = slice(s * 128, (s + 1) * 128)
        d2 = jnp.maximum((qn + pn_ref[:, sl]) + qp[:, sl], 0.0)
        ks = lax.bitcast_convert_type(
            (lax.bitcast_convert_type(d2, jnp.int32) & ~0x3F)
            + (0x00800000 + s), jnp.float32)
        t = jnp.maximum(m1, ks)
        m1 = jnp.minimum(m1, ks)
        u = jnp.maximum(m2, t)
        m2 = jnp.minimum(m2, t)
        m3 = jnp.minimum(m3, u)
    cands = jnp.concatenate([m1, m2, m3], axis=1)
    lane = lax.broadcasted_iota(jnp.int32, (BM, 3 * 128), 1)
    outs, cols = [], []
    work = cands
    for _ in range(KNN):
        mn = jnp.min(work, axis=1, keepdims=True)
        pos = jnp.where(work == mn, lane, jnp.int32(1 << 20))
        pk = jnp.min(pos, axis=1, keepdims=True)
        work = jnp.where(lane == pk, big, work)
        outs.append(mn)
        cols.append(pk & 127)
    wb = (lax.bitcast_convert_type(jnp.concatenate(outs, axis=1), jnp.int32)
          - 0x00800000)
    d2k = lax.bitcast_convert_type(wb & ~0x3F, jnp.float32)
    idx_ref[...] = (wb & 0x3F) * 128 + jnp.concatenate(cols, axis=1)
    w = 1.0 / jnp.maximum(d2k, 1e-16)
    w_ref[...] = w / jnp.sum(w, axis=1, keepdims=True)


_knn_call = pl.pallas_call(
    _knn_body,
    grid=(M // BM,),
    in_specs=[
        pl.BlockSpec((BM, 8), lambda i: (i, 0)),
        pl.BlockSpec((BM, 1), lambda i: (i, 0)),
        pl.BlockSpec((8, N), lambda i: (0, 0)),
        pl.BlockSpec((1, N), lambda i: (0, 0)),
    ],
    out_specs=[
        pl.BlockSpec((BM, KNN), lambda i: (i, 0)),
        pl.BlockSpec((BM, KNN), lambda i: (i, 0)),
    ],
    out_shape=[
        jax.ShapeDtypeStruct((M, KNN), jnp.int32),
        jax.ShapeDtypeStruct((M, KNN), jnp.float32),
    ],
)


_NB = 3


def _sc_gather_body(idx_hbm, tab_hbm, out_hbm, idx_v, rows_v,
                    si0, si1, si2, sg0, sg1, sg2, ss0, ss1, ss2):
    wid = lax.axis_index("s") * _NC + lax.axis_index("c")
    base = wid * _ROWS_PER_W
    sem_i, sem_g, sem_s = (si0, si1, si2), (sg0, sg1, sg2), (ss0, ss1, ss2)
    ci, cg, cs = {}, {}, {}

    def start_idx(c):
        b = c % _NB
        ci[c] = pltpu.async_copy(
            idx_hbm.at[pl.ds(base + c * _GCHUNK, _GCHUNK)],
            idx_v.at[b], sem_i[b])

    for c in range(min(_NB, _NCHUNK)):
        start_idx(c)
    for c in range(_NCHUNK):
        b = c % _NB
        ci[c].wait()
        if c >= _NB:
            cs[c - _NB].wait()
        cg[c] = pltpu.async_copy(tab_hbm.at[idx_v.at[b]], rows_v.at[b],
                                 sem_g[b])
        if c >= 1:
            bp = (c - 1) % _NB
            cg[c - 1].wait()
            cs[c - 1] = pltpu.async_copy(
                rows_v.at[bp],
                out_hbm.at[pl.ds(base + (c - 1) * _GCHUNK, _GCHUNK)],
                sem_s[bp])
            if c + 2 < _NCHUNK:
                start_idx(c + 2)
    c = _NCHUNK - 1
    cg[c].wait()
    cs[c] = pltpu.async_copy(
        rows_v.at[c % _NB],
        out_hbm.at[pl.ds(base + c * _GCHUNK, _GCHUNK)], sem_s[c % _NB])
    for c in range(max(0, _NCHUNK - _NB), _NCHUNK):
        cs[c].wait()


@functools.lru_cache(maxsize=None)
def _sc_gather():
    return pl.kernel(
        _sc_gather_body,
        out_type=jax.ShapeDtypeStruct((_ROWS, C), jnp.float32),
        mesh=plsc.VectorSubcoreMesh(core_axis_name="c", subcore_axis_name="s",
                                    num_cores=_NC, num_subcores=_NS),
        scratch_types=[
            pltpu.VMEM((_NB, _GCHUNK), jnp.int32),
            pltpu.VMEM((_NB, _GCHUNK, C), jnp.float32),
        ] + [pltpu.SemaphoreType.DMA] * (3 * _NB),
        compiler_params=pltpu.CompilerParams(use_tc_tiling_on_sc=False),
    )


def _mlp_body(w_ref, g0_ref, g1_ref, g2_ref, xs_ref, w1t_ref, w2t_ref, b_ref,
              y_ref):
    w = w_ref[...]
    xi = (w[:, 0:1] * g0_ref[...] + w[:, 1:2] * g1_ref[...]
          + w[:, 2:3] * g2_ref[...])
    acc = jnp.dot(xi, w1t_ref[...], preferred_element_type=jnp.float32)
    acc = acc + jnp.dot(xs_ref[...], w2t_ref[...],
                        preferred_element_type=jnp.float32)
    y_ref[...] = jnp.maximum(acc + b_ref[...], 0.0)


_mlp_call = pl.pallas_call(
    _mlp_body,
    grid=(M // BC,),
    in_specs=[
        pl.BlockSpec((BC, KNN), lambda i: (i, 0)),
        pl.BlockSpec((BC, C), lambda i: (i, 0)),
        pl.BlockSpec((BC, C), lambda i: (i + M // BC, 0)),
        pl.BlockSpec((BC, C), lambda i: (i + 2 * (M // BC), 0)),
        pl.BlockSpec((BC, CS), lambda i: (i, 0)),
        pl.BlockSpec((C, DOUT), lambda i: (0, 0)),
        pl.BlockSpec((CS, DOUT), lambda i: (0, 0)),
        pl.BlockSpec((1, DOUT), lambda i: (0, 0)),
    ],
    out_specs=pl.BlockSpec((BC, DOUT), lambda i: (i, 0)),
    out_shape=jax.ShapeDtypeStruct((M, DOUT), jnp.float32),
)


def kernel(x, pos, batch, x_skip, pos_skip, batch_skip, W, b):
    qn = jnp.sum(pos_skip * pos_skip, axis=1, keepdims=True)
    pn = jnp.sum(pos * pos, axis=1)[None, :]
    q_pad = jnp.concatenate(
        [pos_skip, jnp.zeros((M, 5), jnp.float32)], axis=1)
    pt_pad = jnp.concatenate(
        [(-2.0 * pos).T, jnp.zeros((5, N), jnp.float32)], axis=0)
    idx, w = _knn_call(q_pad, qn, pt_pad, pn)

    flat_idx = idx.T.reshape(-1)
    g = _sc_gather()(flat_idx, x)

    w1t = W[:, :C].T
    w2t = W[:, C:].T
    y = _mlp_call(w, g, g, g, x_skip, w1t, w2t, b[None, :])
    return (y, pos_skip, batch_skip)


# speedup vs baseline: 4.9978x; 1.0187x over previous
import functools

import jax
import jax.numpy as jnp
from jax import lax
from jax.experimental import pallas as pl
from jax.experimental.pallas import tpu as pltpu
from jax.experimental.pallas import tpu_sc as plsc

N = 8192
M = 32768
C = 64
CS = 64
DOUT = 128
KNN = 3

BM = 1024
BC = 2048

_NC = 2
_NS = 16
_NW = _NC * _NS
_GCHUNK = 128
_ROWS = KNN * M
_ROWS_PER_W = _ROWS // _NW
_NCHUNK = _ROWS_PER_W // _GCHUNK


def _knn_body(q_ref, qn_ref, pt_ref, pn_ref, idx_ref, w_ref):
    qp = jnp.dot(q_ref[...], pt_ref[...], preferred_element_type=jnp.float32)
    qn = qn_ref[...]
    big = jnp.float32(3.0e38)
    m1 = jnp.full((BM, 128), big, jnp.float32)
    m2 = m1
    m3 = m1
    for s in range(N // 128):
        sl = slice(s * 128, (s + 1) * 128)
        d2 = jnp.maximum((qn + pn_ref[:, sl]) + qp[:, sl], 0.0)
        ks = lax.bitcast_convert_type(
            (lax.bitcast_convert_type(d2, jnp.int32) & ~0x3F)
            + (0x00800000 + s), jnp.float32)
        t = jnp.maximum(m1, ks)
        m1 = jnp.minimum(m1, ks)
        u = jnp.maximum(m2, t)
        m2 = jnp.minimum(m2, t)
        m3 = jnp.minimum(m3, u)
    cands = jnp.concatenate([m1, m2, m3], axis=1)
    lane = lax.broadcasted_iota(jnp.int32, (BM, 3 * 128), 1)
    outs, cols = [], []
    work = cands
    for _ in range(KNN):
        mn = jnp.min(work, axis=1, keepdims=True)
        pos = jnp.where(work == mn, lane, jnp.int32(1 << 20))
        pk = jnp.min(pos, axis=1, keepdims=True)
        work = jnp.where(lane == pk, big, work)
        outs.append(mn)
        cols.append(pk & 127)
    wb = (lax.bitcast_convert_type(jnp.concatenate(outs, axis=1), jnp.int32)
          - 0x00800000)
    d2k = lax.bitcast_convert_type(wb & ~0x3F, jnp.float32)
    idx_ref[...] = (wb & 0x3F) * 128 + jnp.concatenate(cols, axis=1)
    w = 1.0 / jnp.maximum(d2k, 1e-16)
    w_ref[...] = w / jnp.sum(w, axis=1, keepdims=True)


_knn_call = pl.pallas_call(
    _knn_body,
    grid=(M // BM,),
    in_specs=[
        pl.BlockSpec((BM, 8), lambda i: (i, 0)),
        pl.BlockSpec((BM, 1), lambda i: (i, 0)),
        pl.BlockSpec((8, N), lambda i: (0, 0)),
        pl.BlockSpec((1, N), lambda i: (0, 0)),
    ],
    out_specs=[
        pl.BlockSpec((BM, KNN), lambda i: (i, 0)),
        pl.BlockSpec((BM, KNN), lambda i: (i, 0)),
    ],
    out_shape=[
        jax.ShapeDtypeStruct((M, KNN), jnp.int32),
        jax.ShapeDtypeStruct((M, KNN), jnp.float32),
    ],
)


_NB = 3


def _sc_gather_body(idx_hbm, tab_hbm, out_hbm, idx_v, rows_v, shared,
                    si0, si1, si2, sg0, sg1, sg2, ss0, ss1, ss2):
    sid = lax.axis_index("s")
    wid = sid * _NC + lax.axis_index("c")
    base = wid * _ROWS_PER_W
    sem_i, sem_g, sem_s = (si0, si1, si2), (sg0, sg1, sg2), (ss0, ss1, ss2)
    ci, cg, cs = {}, {}, {}

    @pl.when(sid == 0)
    def _stage():
        pltpu.sync_copy(tab_hbm, shared)

    plsc.subcore_barrier()

    def start_idx(c):
        b = c % _NB
        ci[c] = pltpu.async_copy(
            idx_hbm.at[pl.ds(base + c * _GCHUNK, _GCHUNK)],
            idx_v.at[b], sem_i[b])

    for c in range(min(_NB, _NCHUNK)):
        start_idx(c)
    for c in range(_NCHUNK):
        b = c % _NB
        ci[c].wait()
        if c >= _NB:
            cs[c - _NB].wait()
        cg[c] = pltpu.async_copy(shared.at[idx_v.at[b]], rows_v.at[b],
                                 sem_g[b])
        if c >= 1:
            bp = (c - 1) % _NB
            cg[c - 1].wait()
            cs[c - 1] = pltpu.async_copy(
                rows_v.at[bp],
                out_hbm.at[pl.ds(base + (c - 1) * _GCHUNK, _GCHUNK)],
                sem_s[bp])
            if c + 2 < _NCHUNK:
                start_idx(c + 2)
    c = _NCHUNK - 1
    cg[c].wait()
    cs[c] = pltpu.async_copy(
        rows_v.at[c % _NB],
        out_hbm.at[pl.ds(base + c * _GCHUNK, _GCHUNK)], sem_s[c % _NB])
    for c in range(max(0, _NCHUNK - _NB), _NCHUNK):
        cs[c].wait()


@functools.lru_cache(maxsize=None)
def _sc_gather():
    return pl.kernel(
        _sc_gather_body,
        out_type=jax.ShapeDtypeStruct((_ROWS, C), jnp.float32),
        mesh=plsc.VectorSubcoreMesh(core_axis_name="c", subcore_axis_name="s",
                                    num_cores=_NC, num_subcores=_NS),
        scratch_types=[
            pltpu.VMEM((_NB, _GCHUNK), jnp.int32),
            pltpu.VMEM((_NB, _GCHUNK, C), jnp.float32),
            pltpu.VMEM_SHARED((N, C), jnp.float32),
        ] + [pltpu.SemaphoreType.DMA] * (3 * _NB),
        compiler_params=pltpu.CompilerParams(use_tc_tiling_on_sc=False),
    )


def _mlp_body(w_ref, g0_ref, g1_ref, g2_ref, xs_ref, w1t_ref, w2t_ref, b_ref,
              y_ref):
    w = w_ref[...]
    xi = (w[:, 0:1] * g0_ref[...] + w[:, 1:2] * g1_ref[...]
          + w[:, 2:3] * g2_ref[...])
    acc = jnp.dot(xi, w1t_ref[...], preferred_element_type=jnp.float32)
    acc = acc + jnp.dot(xs_ref[...], w2t_ref[...],
                        preferred_element_type=jnp.float32)
    y_ref[...] = jnp.maximum(acc + b_ref[...], 0.0)


_mlp_call = pl.pallas_call(
    _mlp_body,
    grid=(M // BC,),
    in_specs=[
        pl.BlockSpec((BC, KNN), lambda i: (i, 0)),
        pl.BlockSpec((BC, C), lambda i: (i, 0)),
        pl.BlockSpec((BC, C), lambda i: (i + M // BC, 0)),
        pl.BlockSpec((BC, C), lambda i: (i + 2 * (M // BC), 0)),
        pl.BlockSpec((BC, CS), lambda i: (i, 0)),
        pl.BlockSpec((C, DOUT), lambda i: (0, 0)),
        pl.BlockSpec((CS, DOUT), lambda i: (0, 0)),
        pl.BlockSpec((1, DOUT), lambda i: (0, 0)),
    ],
    out_specs=pl.BlockSpec((BC, DOUT), lambda i: (i, 0)),
    out_shape=jax.ShapeDtypeStruct((M, DOUT), jnp.float32),
)


def kernel(x, pos, batch, x_skip, pos_skip, batch_skip, W, b):
    qn = jnp.sum(pos_skip * pos_skip, axis=1, keepdims=True)
    pn = jnp.sum(pos * pos, axis=1)[None, :]
    q_pad = jnp.concatenate(
        [pos_skip, jnp.zeros((M, 5), jnp.float32)], axis=1)
    pt_pad = jnp.concatenate(
        [(-2.0 * pos).T, jnp.zeros((5, N), jnp.float32)], axis=0)
    idx, w = _knn_call(q_pad, qn, pt_pad, pn)

    flat_idx = idx.T.reshape(-1)
    g = _sc_gather()(flat_idx, x)

    w1t = W[:, :C].T
    w2t = W[:, C:].T
    y = _mlp_call(w, g, g, g, x_skip, w1t, w2t, b[None, :])
    return (y, pos_skip, batch_skip)
